# Initial kernel scaffold; baseline (speedup 1.0000x reference)
import jax
import jax.numpy as jnp
from jax.experimental import pallas as pl


def kernel(x, edge_index, W1_l, b1, W1_r, W2_l, b2, W2_r, W3_l, b3, W3_r):
    raise NotImplementedError("write your pallas kernel here")


# speedup vs baseline: 6.2531x; 6.2531x over previous
import jax
import jax.numpy as jnp
from jax import lax
from jax.experimental import pallas as pl
from jax.experimental.pallas import tpu as pltpu
from jax.experimental.pallas import tpu_sc as plsc

N = 10000
NC = 2
NS = 16
NW = NC * NS
C = 128
CHUNKS = 80
EPT = C * CHUNKS
E_PAD = NW * EPT
R = 10240
RPT = R // NS


def _make_sc_agg(D, with_counts):
    mesh = plsc.VectorSubcoreMesh(core_axis_name="c", subcore_axis_name="s")
    out_type = [jax.ShapeDtypeStruct((NC, R, D), jnp.float32)]
    scratch = [
        pltpu.VMEM_SHARED((R, D), jnp.float32),
        pltpu.VMEM((C,), jnp.int32),
        pltpu.VMEM((C,), jnp.int32),
        pltpu.VMEM((C, D), jnp.float32),
        pltpu.SemaphoreType.DMA,
    ]
    if with_counts:
        out_type.append(jax.ShapeDtypeStruct((NC * R,), jnp.float32))
        scratch += [
            pltpu.VMEM_SHARED((R,), jnp.float32),
            pltpu.VMEM((C,), jnp.float32),
        ]

    def body(x_hbm, src_hbm, dst_hbm, zf_hbm, *rest):
        if with_counts:
            (zc_hbm, ones_hbm, out_hbm, cnt_hbm,
             acc, sidx, didx, gbuf, sem, cacc, ones_v) = rest
        else:
            (out_hbm, acc, sidx, didx, gbuf, sem) = rest
        c = lax.axis_index("c")
        s = lax.axis_index("s")
        row0 = s * RPT
        pltpu.sync_copy(zf_hbm, gbuf)
        for k in range(RPT // C):
            pltpu.sync_copy(gbuf, acc.at[pl.ds(row0 + k * C, C)])
        if with_counts:
            pltpu.sync_copy(zc_hbm, ones_v)
            for k in range(RPT // C):
                pltpu.sync_copy(ones_v, cacc.at[pl.ds(row0 + k * C, C)])
            pltpu.sync_copy(ones_hbm, ones_v)
        plsc.subcore_barrier()
        base = (c * NS + s) * EPT

        def step(i, carry):
            off = base + i * C
            pltpu.sync_copy(src_hbm.at[pl.ds(off, C)], sidx)
            pltpu.sync_copy(dst_hbm.at[pl.ds(off, C)], didx)
            pltpu.async_copy(x_hbm.at[sidx], gbuf, sem).wait()
            pltpu.sync_copy(gbuf, acc.at[didx], add=True)
            if with_counts:
                pltpu.sync_copy(ones_v, cacc.at[didx], add=True)
            return carry

        lax.fori_loop(0, CHUNKS, step, 0)
        plsc.subcore_barrier()
        for k in range(RPT // C):
            pltpu.sync_copy(acc.at[pl.ds(row0 + k * C, C)], gbuf)
            pltpu.sync_copy(gbuf, out_hbm.at[c, pl.ds(row0 + k * C, C)])
        if with_counts:
            for k in range(RPT // C):
                pltpu.sync_copy(cacc.at[pl.ds(row0 + k * C, C)], ones_v)
                pltpu.sync_copy(ones_v,
                                cnt_hbm.at[pl.ds(c * R + row0 + k * C, C)])

    return pl.kernel(body, out_type=out_type, scratch_types=scratch, mesh=mesh)


_sc_agg128c = _make_sc_agg(128, True)
_sc_agg128 = _make_sc_agg(128, False)

_BLK = 1000
_GRID = N // _BLK


def _row_spec(d):
    return pl.BlockSpec((_BLK, d), lambda i: (i, 0))


def _part_spec(d, core):
    return pl.BlockSpec((1, _BLK, d), lambda i, c=core: (c, i, 0))


def _full_spec(shape):
    nd = len(shape)
    return pl.BlockSpec(shape, lambda i: (0,) * nd)


def _mean(pa, pb, inv):
    return (pa[0] + pb[0]) * inv[...]


def _tc1_body(x, pa, pb, cn, wl, b, wr, h1_ref):
    mean = _mean(pa, pb, cn)
    h1_ref[...] = (jnp.dot(mean, wl[...], preferred_element_type=jnp.float32)
                   + b[...]
                   + jnp.dot(x[...], wr[...],
                             preferred_element_type=jnp.float32))


def _tc2_body(h1, pa, pb, cn, wl, b, wr, w3, h2_ref, p3_ref):
    mean = _mean(pa, pb, cn)
    h2 = (jnp.dot(mean, wl[...], preferred_element_type=jnp.float32)
          + b[...]
          + jnp.dot(h1[...], wr[...], preferred_element_type=jnp.float32))
    h2_ref[...] = h2
    p3_ref[...] = jnp.dot(h2, w3[...], preferred_element_type=jnp.float32)


def _tc3_body(h2, pa, pb, cn, b, wr, h3_ref):
    mean = _mean(pa, pb, cn)
    h3_ref[...] = (mean + b[...]
                   + jnp.dot(h2[...], wr[...],
                             preferred_element_type=jnp.float32))


def kernel(x, edge_index, W1_l, b1, W1_r, W2_l, b2, W2_r, W3_l, b3, W3_r):
    src = edge_index[0].astype(jnp.int32)
    dst = edge_index[1].astype(jnp.int32)
    pad = E_PAD - src.shape[0]
    pidx = jnp.arange(pad, dtype=jnp.int32)
    src_p = jnp.concatenate([src, (pidx * 131) % N])
    dst_p = jnp.concatenate([dst, N + pidx % (R - N)])

    zf128 = jnp.zeros((C, 128), jnp.float32)
    zc1 = jnp.zeros((C,), jnp.float32)
    ones1 = jnp.ones((C,), jnp.float32)

    w1l = W1_l.T
    w1r = W1_r.T
    w2l = W2_l.T
    w2r = W2_r.T
    w3l = W3_l.T
    w3r = W3_r.T
    b1r = b1.reshape(1, -1)
    b2r = b2.reshape(1, -1)
    b3r = b3.reshape(1, -1)

    agg1, cnt_flat = _sc_agg128c(x, src_p, dst_p, zf128, zc1, ones1)
    cnt_tot = cnt_flat[:N] + cnt_flat[R:R + N]
    inv = (1.0 / jnp.maximum(cnt_tot, 1.0))[:, None]

    cnt_spec = pl.BlockSpec((_BLK, 1), lambda i: (i, 0))

    h1 = pl.pallas_call(
        _tc1_body,
        grid=(_GRID,),
        in_specs=[_row_spec(128), _part_spec(128, 0), _part_spec(128, 1),
                  cnt_spec,
                  _full_spec((128, 128)), _full_spec((1, 128)),
                  _full_spec((128, 128))],
        out_specs=_row_spec(128),
        out_shape=jax.ShapeDtypeStruct((N, 128), jnp.float32),
    )(x, agg1, agg1, inv, w1l, b1r, w1r)

    (agg2,) = _sc_agg128(h1, src_p, dst_p, zf128)

    h2, p3 = pl.pallas_call(
        _tc2_body,
        grid=(_GRID,),
        in_specs=[_row_spec(128), _part_spec(128, 0), _part_spec(128, 1),
                  cnt_spec,
                  _full_spec((128, 64)), _full_spec((1, 64)),
                  _full_spec((128, 64)), _full_spec((64, 128))],
        out_specs=[_row_spec(64), _row_spec(128)],
        out_shape=[jax.ShapeDtypeStruct((N, 64), jnp.float32),
                   jax.ShapeDtypeStruct((N, 128), jnp.float32)],
    )(h1, agg2, agg2, inv, w2l, b2r, w2r, w3l)

    (agg3,) = _sc_agg128(p3, src_p, dst_p, zf128)

    h3 = pl.pallas_call(
        _tc3_body,
        grid=(_GRID,),
        in_specs=[_row_spec(64), _part_spec(128, 0), _part_spec(128, 1),
                  cnt_spec,
                  _full_spec((1, 128)), _full_spec((64, 128))],
        out_specs=_row_spec(128),
        out_shape=jax.ShapeDtypeStruct((N, 128), jnp.float32),
    )(h2, agg3, agg3, inv, b3r, w3r)

    return h3


# speedup vs baseline: 11.3286x; 1.8117x over previous
import jax
import jax.numpy as jnp
from jax import lax
from jax.experimental import pallas as pl
from jax.experimental.pallas import tpu as pltpu
from jax.experimental.pallas import tpu_sc as plsc

N = 10000
NC = 2
NS = 16
NW = NC * NS
C = 128
CHUNKS = 80
EPT = C * CHUNKS
E_PAD = NW * EPT
R = 10240
RPT = R // NS


def _make_sc_agg(D, with_counts):
    mesh = plsc.VectorSubcoreMesh(core_axis_name="c", subcore_axis_name="s")
    out_type = [jax.ShapeDtypeStruct((NC, R, D), jnp.float32)]
    scratch = [
        pltpu.VMEM_SHARED((R, D), jnp.float32),
        pltpu.VMEM((2, C), jnp.int32),
        pltpu.VMEM((2, C), jnp.int32),
        pltpu.VMEM((C, D), jnp.float32),
        pltpu.VMEM((C, D), jnp.float32),
        pltpu.SemaphoreType.DMA,
        pltpu.SemaphoreType.DMA,
    ]
    if with_counts:
        out_type.append(jax.ShapeDtypeStruct((NC * R,), jnp.float32))
        scratch += [
            pltpu.VMEM_SHARED((R,), jnp.float32),
            pltpu.VMEM((C,), jnp.float32),
        ]

    def body(x_hbm, edges_hbm, zf_hbm, *rest):
        if with_counts:
            (zc_hbm, ones_hbm, out_hbm, cnt_hbm,
             acc, ib0, ib1, gb0, gb1, sem0, sem1, cacc, ones_v) = rest
        else:
            (out_hbm, acc, ib0, ib1, gb0, gb1, sem0, sem1) = rest
        ibs, gbs, sems = (ib0, ib1), (gb0, gb1), (sem0, sem1)
        c = lax.axis_index("c")
        s = lax.axis_index("s")
        row0 = s * RPT
        pltpu.sync_copy(zf_hbm, gb0)
        for k in range(RPT // C):
            pltpu.sync_copy(gb0, acc.at[pl.ds(row0 + k * C, C)])
        if with_counts:
            pltpu.sync_copy(zc_hbm, ones_v)
            for k in range(RPT // C):
                pltpu.sync_copy(ones_v, cacc.at[pl.ds(row0 + k * C, C)])
            pltpu.sync_copy(ones_hbm, ones_v)
        plsc.subcore_barrier()
        cbase = (c * NS + s) * CHUNKS

        def drain_scatter_refill(b, cid, refill):
            pltpu.make_async_copy(
                x_hbm.at[ibs[b].at[0]], gbs[b], sems[b]).wait()
            pltpu.sync_copy(gbs[b], acc.at[ibs[b].at[1]], add=True)
            if with_counts:
                pltpu.sync_copy(ones_v, cacc.at[ibs[b].at[1]], add=True)
            if refill:
                pltpu.sync_copy(edges_hbm.at[cid], ibs[b])
                pltpu.async_copy(x_hbm.at[ibs[b].at[0]], gbs[b], sems[b])

        for b in range(2):
            pltpu.sync_copy(edges_hbm.at[cbase + b], ibs[b])
            pltpu.async_copy(x_hbm.at[ibs[b].at[0]], gbs[b], sems[b])

        def step(j, carry):
            for b in range(2):
                drain_scatter_refill(b, cbase + 2 * j + b + 2, True)
            return carry

        lax.fori_loop(0, CHUNKS // 2 - 1, step, 0)
        for b in range(2):
            drain_scatter_refill(b, 0, False)
        plsc.subcore_barrier()
        for k in range(RPT // C):
            pltpu.sync_copy(acc.at[pl.ds(row0 + k * C, C)], gb0)
            pltpu.sync_copy(gb0, out_hbm.at[c, pl.ds(row0 + k * C, C)])
        if with_counts:
            for k in range(RPT // C):
                pltpu.sync_copy(cacc.at[pl.ds(row0 + k * C, C)], ones_v)
                pltpu.sync_copy(ones_v,
                                cnt_hbm.at[pl.ds(c * R + row0 + k * C, C)])

    return pl.kernel(body, out_type=out_type, scratch_types=scratch, mesh=mesh)


_sc_agg128c = _make_sc_agg(128, True)
_sc_agg128 = _make_sc_agg(128, False)

_BLK = 1000
_GRID = N // _BLK


def _row_spec(d):
    return pl.BlockSpec((_BLK, d), lambda i: (i, 0))


def _part_spec(d, core):
    return pl.BlockSpec((1, _BLK, d), lambda i, c=core: (c, i, 0))


def _full_spec(shape):
    nd = len(shape)
    return pl.BlockSpec(shape, lambda i: (0,) * nd)


def _mean(pa, pb, inv):
    return (pa[0] + pb[0]) * inv[...]


def _tc1_body(x, pa, pb, cn, wl, b, wr, h1_ref):
    mean = _mean(pa, pb, cn)
    h1_ref[...] = (jnp.dot(mean, wl[...], preferred_element_type=jnp.float32)
                   + b[...]
                   + jnp.dot(x[...], wr[...],
                             preferred_element_type=jnp.float32))


def _tc2_body(h1, pa, pb, cn, wl, b, wr, w3, h2_ref, p3_ref):
    mean = _mean(pa, pb, cn)
    h2 = (jnp.dot(mean, wl[...], preferred_element_type=jnp.float32)
          + b[...]
          + jnp.dot(h1[...], wr[...], preferred_element_type=jnp.float32))
    h2_ref[...] = h2
    p3_ref[...] = jnp.dot(h2, w3[...], preferred_element_type=jnp.float32)


def _tc3_body(h2, pa, pb, cn, b, wr, h3_ref):
    mean = _mean(pa, pb, cn)
    h3_ref[...] = (mean + b[...]
                   + jnp.dot(h2[...], wr[...],
                             preferred_element_type=jnp.float32))


def kernel(x, edge_index, W1_l, b1, W1_r, W2_l, b2, W2_r, W3_l, b3, W3_r):
    src = edge_index[0].astype(jnp.int32)
    dst = edge_index[1].astype(jnp.int32)
    pad = E_PAD - src.shape[0]
    pidx = jnp.arange(pad, dtype=jnp.int32)
    src_p = jnp.concatenate([src, (pidx * 131) % N])
    dst_p = jnp.concatenate([dst, N + pidx % (R - N)])
    edges = jnp.stack([src_p.reshape(-1, C), dst_p.reshape(-1, C)], axis=1)

    zf128 = jnp.zeros((C, 128), jnp.float32)
    zc1 = jnp.zeros((C,), jnp.float32)
    ones1 = jnp.ones((C,), jnp.float32)

    w1l = W1_l.T
    w1r = W1_r.T
    w2l = W2_l.T
    w2r = W2_r.T
    w3l = W3_l.T
    w3r = W3_r.T
    b1r = b1.reshape(1, -1)
    b2r = b2.reshape(1, -1)
    b3r = b3.reshape(1, -1)

    agg1, cnt_flat = _sc_agg128c(x, edges, zf128, zc1, ones1)
    cnt_tot = cnt_flat[:N] + cnt_flat[R:R + N]
    inv = (1.0 / jnp.maximum(cnt_tot, 1.0))[:, None]

    cnt_spec = pl.BlockSpec((_BLK, 1), lambda i: (i, 0))

    h1 = pl.pallas_call(
        _tc1_body,
        grid=(_GRID,),
        in_specs=[_row_spec(128), _part_spec(128, 0), _part_spec(128, 1),
                  cnt_spec,
                  _full_spec((128, 128)), _full_spec((1, 128)),
                  _full_spec((128, 128))],
        out_specs=_row_spec(128),
        out_shape=jax.ShapeDtypeStruct((N, 128), jnp.float32),
    )(x, agg1, agg1, inv, w1l, b1r, w1r)

    (agg2,) = _sc_agg128(h1, edges, zf128)

    h2, p3 = pl.pallas_call(
        _tc2_body,
        grid=(_GRID,),
        in_specs=[_row_spec(128), _part_spec(128, 0), _part_spec(128, 1),
                  cnt_spec,
                  _full_spec((128, 64)), _full_spec((1, 64)),
                  _full_spec((128, 64)), _full_spec((64, 128))],
        out_specs=[_row_spec(64), _row_spec(128)],
        out_shape=[jax.ShapeDtypeStruct((N, 64), jnp.float32),
                   jax.ShapeDtypeStruct((N, 128), jnp.float32)],
    )(h1, agg2, agg2, inv, w2l, b2r, w2r, w3l)

    (agg3,) = _sc_agg128(p3, edges, zf128)

    h3 = pl.pallas_call(
        _tc3_body,
        grid=(_GRID,),
        in_specs=[_row_spec(64), _part_spec(128, 0), _part_spec(128, 1),
                  cnt_spec,
                  _full_spec((1, 128)), _full_spec((64, 128))],
        out_specs=_row_spec(128),
        out_shape=jax.ShapeDtypeStruct((N, 128), jnp.float32),
    )(h2, agg3, agg3, inv, b3r, w3r)

    return h3


# speedup vs baseline: 12.7612x; 1.1265x over previous
import jax
import jax.numpy as jnp
from jax import lax
from jax.experimental import pallas as pl
from jax.experimental.pallas import tpu as pltpu
from jax.experimental.pallas import tpu_sc as plsc

N = 10000
NC = 2
NS = 16
NW = NC * NS
C = 128
NGB = 2
NIB = 4
CHUNKS = 80
EPT = C * CHUNKS
E_PAD = NW * EPT
R = 10240
RPT = R // NS


def _make_sc_agg(D, with_counts):
    mesh = plsc.VectorSubcoreMesh(core_axis_name="c", subcore_axis_name="s")
    out_type = [jax.ShapeDtypeStruct((NC, R, D), jnp.float32)]
    scratch = (
        [pltpu.VMEM_SHARED((R, D), jnp.float32)]
        + [pltpu.VMEM((2, C), jnp.int32)] * NIB
        + [pltpu.VMEM((C, D), jnp.float32)] * NGB
        + [pltpu.SemaphoreType.DMA] * NIB
        + [pltpu.SemaphoreType.DMA] * NGB
    )
    if with_counts:
        out_type.append(jax.ShapeDtypeStruct((NC * R,), jnp.float32))
        scratch += [
            pltpu.VMEM_SHARED((R,), jnp.float32),
            pltpu.VMEM((C,), jnp.float32),
        ]

    def body(x_hbm, edges_hbm, zf_hbm, *rest):
        if with_counts:
            (zc_hbm, ones_hbm, out_hbm, cnt_hbm, acc, *bufs) = rest
            *bufs, cacc, ones_v = bufs
        else:
            (out_hbm, acc, *bufs) = rest
        ibs = tuple(bufs[:NIB])
        gbs = tuple(bufs[NIB:NIB + NGB])
        isems = tuple(bufs[NIB + NGB:2 * NIB + NGB])
        gsems = tuple(bufs[2 * NIB + NGB:2 * NIB + 2 * NGB])
        c = lax.axis_index("c")
        s = lax.axis_index("s")
        row0 = s * RPT
        cbase = (c * NS + s) * CHUNKS

        for b in range(NIB):
            pltpu.async_copy(edges_hbm.at[cbase + b], ibs[b], isems[b])
        pltpu.sync_copy(zf_hbm, gbs[0])
        for k in range(RPT // C):
            pltpu.sync_copy(gbs[0], acc.at[pl.ds(row0 + k * C, C)])
        if with_counts:
            pltpu.sync_copy(zc_hbm, ones_v)
            for k in range(RPT // C):
                pltpu.sync_copy(ones_v, cacc.at[pl.ds(row0 + k * C, C)])
            pltpu.sync_copy(ones_hbm, ones_v)
        plsc.subcore_barrier()

        def idx_wait(q):
            pltpu.make_async_copy(edges_hbm.at[0], ibs[q], isems[q]).wait()

        def gather_start(q, g):
            pltpu.async_copy(x_hbm.at[ibs[q].at[0]], gbs[g], gsems[g])

        def process(b, cid_next, start_next, refill):
            g = b % NGB
            pltpu.make_async_copy(
                x_hbm.at[ibs[b].at[0]], gbs[g], gsems[g]).wait()
            pltpu.sync_copy(gbs[g], acc.at[ibs[b].at[1]], add=True)
            if with_counts:
                pltpu.sync_copy(ones_v, cacc.at[ibs[b].at[1]], add=True)
            if refill:
                pltpu.async_copy(edges_hbm.at[cid_next], ibs[b], isems[b])
            if start_next:
                q = (b + NGB) % NIB
                idx_wait(q)
                gather_start(q, g)

        for g in range(NGB):
            idx_wait(g)
            gather_start(g, g)

        def step(j, carry):
            for b in range(NIB):
                process(b, cbase + NIB * j + b + NIB, True, True)
            return carry

        lax.fori_loop(0, CHUNKS // NIB - 1, step, 0)
        for b in range(NIB):
            process(b, 0, b < NGB, False)
        plsc.subcore_barrier()
        for k in range(RPT // C):
            pltpu.sync_copy(acc.at[pl.ds(row0 + k * C, C)], gbs[0])
            pltpu.sync_copy(gbs[0], out_hbm.at[c, pl.ds(row0 + k * C, C)])
        if with_counts:
            for k in range(RPT // C):
                pltpu.sync_copy(cacc.at[pl.ds(row0 + k * C, C)], ones_v)
                pltpu.sync_copy(ones_v,
                                cnt_hbm.at[pl.ds(c * R + row0 + k * C, C)])

    return pl.kernel(body, out_type=out_type, scratch_types=scratch, mesh=mesh)


_sc_agg128c = _make_sc_agg(128, True)
_sc_agg128 = _make_sc_agg(128, False)

_BLK = 1000
_GRID = N // _BLK


def _row_spec(d):
    return pl.BlockSpec((_BLK, d), lambda i: (i, 0))


def _part_spec(d, core):
    return pl.BlockSpec((1, _BLK, d), lambda i, c=core: (c, i, 0))


def _full_spec(shape):
    nd = len(shape)
    return pl.BlockSpec(shape, lambda i: (0,) * nd)


def _mean(pa, pb, inv):
    return (pa[0] + pb[0]) * inv[...]


def _tc1_body(x, pa, pb, cn, wl, b, wr, h1_ref):
    mean = _mean(pa, pb, cn)
    h1_ref[...] = (jnp.dot(mean, wl[...], preferred_element_type=jnp.float32)
                   + b[...]
                   + jnp.dot(x[...], wr[...],
                             preferred_element_type=jnp.float32))


def _tc2_body(h1, pa, pb, cn, wl, b, wr, w3, h2_ref, p3_ref):
    mean = _mean(pa, pb, cn)
    h2 = (jnp.dot(mean, wl[...], preferred_element_type=jnp.float32)
          + b[...]
          + jnp.dot(h1[...], wr[...], preferred_element_type=jnp.float32))
    h2_ref[...] = h2
    p3_ref[...] = jnp.dot(h2, w3[...], preferred_element_type=jnp.float32)


def _tc3_body(h2, pa, pb, cn, b, wr, h3_ref):
    mean = _mean(pa, pb, cn)
    h3_ref[...] = (mean + b[...]
                   + jnp.dot(h2[...], wr[...],
                             preferred_element_type=jnp.float32))


def kernel(x, edge_index, W1_l, b1, W1_r, W2_l, b2, W2_r, W3_l, b3, W3_r):
    src = edge_index[0].astype(jnp.int32)
    dst = edge_index[1].astype(jnp.int32)
    pad = E_PAD - src.shape[0]
    pidx = jnp.arange(pad, dtype=jnp.int32)
    src_p = jnp.concatenate([src, (pidx * 131) % N])
    dst_p = jnp.concatenate([dst, N + pidx % (R - N)])
    edges = jnp.stack([src_p.reshape(-1, C), dst_p.reshape(-1, C)], axis=1)

    zf128 = jnp.zeros((C, 128), jnp.float32)
    zc1 = jnp.zeros((C,), jnp.float32)
    ones1 = jnp.ones((C,), jnp.float32)

    w1l = W1_l.T
    w1r = W1_r.T
    w2l = W2_l.T
    w2r = W2_r.T
    w3l = W3_l.T
    w3r = W3_r.T
    b1r = b1.reshape(1, -1)
    b2r = b2.reshape(1, -1)
    b3r = b3.reshape(1, -1)

    agg1, cnt_flat = _sc_agg128c(x, edges, zf128, zc1, ones1)
    cnt_tot = cnt_flat[:N] + cnt_flat[R:R + N]
    inv = (1.0 / jnp.maximum(cnt_tot, 1.0))[:, None]

    cnt_spec = pl.BlockSpec((_BLK, 1), lambda i: (i, 0))

    h1 = pl.pallas_call(
        _tc1_body,
        grid=(_GRID,),
        in_specs=[_row_spec(128), _part_spec(128, 0), _part_spec(128, 1),
                  cnt_spec,
                  _full_spec((128, 128)), _full_spec((1, 128)),
                  _full_spec((128, 128))],
        out_specs=_row_spec(128),
        out_shape=jax.ShapeDtypeStruct((N, 128), jnp.float32),
    )(x, agg1, agg1, inv, w1l, b1r, w1r)

    (agg2,) = _sc_agg128(h1, edges, zf128)

    h2, p3 = pl.pallas_call(
        _tc2_body,
        grid=(_GRID,),
        in_specs=[_row_spec(128), _part_spec(128, 0), _part_spec(128, 1),
                  cnt_spec,
                  _full_spec((128, 64)), _full_spec((1, 64)),
                  _full_spec((128, 64)), _full_spec((64, 128))],
        out_specs=[_row_spec(64), _row_spec(128)],
        out_shape=[jax.ShapeDtypeStruct((N, 64), jnp.float32),
                   jax.ShapeDtypeStruct((N, 128), jnp.float32)],
    )(h1, agg2, agg2, inv, w2l, b2r, w2r, w3l)

    (agg3,) = _sc_agg128(p3, edges, zf128)

    h3 = pl.pallas_call(
        _tc3_body,
        grid=(_GRID,),
        in_specs=[_row_spec(64), _part_spec(128, 0), _part_spec(128, 1),
                  cnt_spec,
                  _full_spec((1, 128)), _full_spec((64, 128))],
        out_specs=_row_spec(128),
        out_shape=jax.ShapeDtypeStruct((N, 128), jnp.float32),
    )(h2, agg3, agg3, inv, b3r, w3r)

    return h3


# speedup vs baseline: 13.0274x; 1.0209x over previous
import jax
import jax.numpy as jnp
from jax import lax
from jax.experimental import pallas as pl
from jax.experimental.pallas import tpu as pltpu
from jax.experimental.pallas import tpu_sc as plsc

N = 10000
NC = 2
NS = 16
NW = NC * NS
C = 128
NGB = 2
NIB = 4
ZC = 64
CHUNKS = 80
EPT = C * CHUNKS
E_PAD = NW * EPT
R = 10240
RPT = R // NS


def _make_sc_agg(D, with_counts):
    mesh = plsc.VectorSubcoreMesh(core_axis_name="c", subcore_axis_name="s")
    out_type = [jax.ShapeDtypeStruct((NC, R, D), jnp.float32)]
    scratch = (
        [pltpu.VMEM_SHARED((R, D), jnp.float32)]
        + [pltpu.VMEM((2, C), jnp.int32)] * NIB
        + [pltpu.VMEM((C, D), jnp.float32)] * NGB
        + [pltpu.SemaphoreType.DMA] * NIB
        + [pltpu.SemaphoreType.DMA] * NGB
        + [pltpu.VMEM((ZC, D), jnp.float32)]
    )
    if with_counts:
        out_type.append(jax.ShapeDtypeStruct((NC * R,), jnp.float32))
        scratch += [
            pltpu.VMEM_SHARED((R,), jnp.float32),
            pltpu.VMEM((C,), jnp.float32),
        ]

    def body(x_hbm, edges_hbm, zf_hbm, *rest):
        if with_counts:
            (zc_hbm, ones_hbm, out_hbm, cnt_hbm, acc, *bufs) = rest
            *bufs, cacc, ones_v = bufs
        else:
            (out_hbm, acc, *bufs) = rest
        ibs = tuple(bufs[:NIB])
        gbs = tuple(bufs[NIB:NIB + NGB])
        isems = tuple(bufs[NIB + NGB:2 * NIB + NGB])
        gsems = tuple(bufs[2 * NIB + NGB:2 * NIB + 2 * NGB])
        zbuf = bufs[2 * NIB + 2 * NGB]
        c = lax.axis_index("c")
        s = lax.axis_index("s")
        row0 = s * RPT
        cbase = (c * NS + s) * CHUNKS

        def idx_wait(q):
            pltpu.make_async_copy(edges_hbm.at[0], ibs[q], isems[q]).wait()

        def gather_start(q, g):
            pltpu.async_copy(x_hbm.at[ibs[q].at[0]], gbs[g], gsems[g])

        for b in range(NIB):
            pltpu.async_copy(edges_hbm.at[cbase + b], ibs[b], isems[b])
        for g in range(NGB):
            idx_wait(g)
            gather_start(g, g)
        pltpu.sync_copy(zf_hbm, zbuf)
        for k in range(RPT // ZC):
            pltpu.sync_copy(zbuf, acc.at[pl.ds(row0 + k * ZC, ZC)])
        if with_counts:
            pltpu.sync_copy(zc_hbm, ones_v)
            for k in range(RPT // C):
                pltpu.sync_copy(ones_v, cacc.at[pl.ds(row0 + k * C, C)])
            pltpu.sync_copy(ones_hbm, ones_v)
        plsc.subcore_barrier()

        def process(b, cid_next, start_next, refill):
            g = b % NGB
            pltpu.make_async_copy(
                x_hbm.at[ibs[b].at[0]], gbs[g], gsems[g]).wait()
            pltpu.sync_copy(gbs[g], acc.at[ibs[b].at[1]], add=True)
            if with_counts:
                pltpu.sync_copy(ones_v, cacc.at[ibs[b].at[1]], add=True)
            if refill:
                pltpu.async_copy(edges_hbm.at[cid_next], ibs[b], isems[b])
            if start_next:
                q = (b + NGB) % NIB
                idx_wait(q)
                gather_start(q, g)

        def step(j, carry):
            for b in range(NIB):
                process(b, cbase + NIB * j + b + NIB, True, True)
            return carry

        lax.fori_loop(0, CHUNKS // NIB - 1, step, 0)
        for b in range(NIB):
            process(b, 0, b < NGB, False)
        plsc.subcore_barrier()
        for k in range(RPT // C):
            g = k % NGB
            if k >= NGB:
                pltpu.make_async_copy(
                    gbs[g], out_hbm.at[c, pl.ds(row0, C)], gsems[g]).wait()
            pltpu.sync_copy(acc.at[pl.ds(row0 + k * C, C)], gbs[g])
            pltpu.async_copy(gbs[g], out_hbm.at[c, pl.ds(row0 + k * C, C)],
                             gsems[g])
        for g in range(NGB):
            pltpu.make_async_copy(
                gbs[g], out_hbm.at[c, pl.ds(row0, C)], gsems[g]).wait()
        if with_counts:
            for k in range(RPT // C):
                pltpu.sync_copy(cacc.at[pl.ds(row0 + k * C, C)], ones_v)
                pltpu.sync_copy(ones_v,
                                cnt_hbm.at[pl.ds(c * R + row0 + k * C, C)])

    return pl.kernel(body, out_type=out_type, scratch_types=scratch, mesh=mesh)


_sc_agg128c = _make_sc_agg(128, True)
_sc_agg128 = _make_sc_agg(128, False)

_BLK = 1000
_GRID = N // _BLK


def _row_spec(d):
    return pl.BlockSpec((_BLK, d), lambda i: (i, 0))


def _part_spec(d, core):
    return pl.BlockSpec((1, _BLK, d), lambda i, c=core: (c, i, 0))


def _full_spec(shape):
    nd = len(shape)
    return pl.BlockSpec(shape, lambda i: (0,) * nd)


def _mean(pa, pb, inv):
    return (pa[0] + pb[0]) * inv[...]


def _tc1_body(x, pa, pb, cn, wl, b, wr, h1_ref):
    mean = _mean(pa, pb, cn)
    h1_ref[...] = (jnp.dot(mean, wl[...], preferred_element_type=jnp.float32)
                   + b[...]
                   + jnp.dot(x[...], wr[...],
                             preferred_element_type=jnp.float32))


def _tc2_body(h1, pa, pb, cn, wl, b, wr, w3, h2_ref, p3_ref):
    mean = _mean(pa, pb, cn)
    h2 = (jnp.dot(mean, wl[...], preferred_element_type=jnp.float32)
          + b[...]
          + jnp.dot(h1[...], wr[...], preferred_element_type=jnp.float32))
    h2_ref[...] = h2
    p3_ref[...] = jnp.dot(h2, w3[...], preferred_element_type=jnp.float32)


def _tc3_body(h2, pa, pb, cn, b, wr, h3_ref):
    mean = _mean(pa, pb, cn)
    h3_ref[...] = (mean + b[...]
                   + jnp.dot(h2[...], wr[...],
                             preferred_element_type=jnp.float32))


def kernel(x, edge_index, W1_l, b1, W1_r, W2_l, b2, W2_r, W3_l, b3, W3_r):
    src = edge_index[0].astype(jnp.int32)
    dst = edge_index[1].astype(jnp.int32)
    pad = E_PAD - src.shape[0]
    pidx = jnp.arange(pad, dtype=jnp.int32)
    src_p = jnp.concatenate([src, (pidx * 131) % N])
    dst_p = jnp.concatenate([dst, N + pidx % (R - N)])
    edges = jnp.stack([src_p.reshape(-1, C), dst_p.reshape(-1, C)], axis=1)

    zf128 = jnp.zeros((ZC, 128), jnp.float32)
    zc1 = jnp.zeros((C,), jnp.float32)
    ones1 = jnp.ones((C,), jnp.float32)

    w1l = W1_l.T
    w1r = W1_r.T
    w2l = W2_l.T
    w2r = W2_r.T
    w3l = W3_l.T
    w3r = W3_r.T
    b1r = b1.reshape(1, -1)
    b2r = b2.reshape(1, -1)
    b3r = b3.reshape(1, -1)

    agg1, cnt_flat = _sc_agg128c(x, edges, zf128, zc1, ones1)
    cnt_tot = cnt_flat[:N] + cnt_flat[R:R + N]
    inv = (1.0 / jnp.maximum(cnt_tot, 1.0))[:, None]

    cnt_spec = pl.BlockSpec((_BLK, 1), lambda i: (i, 0))

    h1 = pl.pallas_call(
        _tc1_body,
        grid=(_GRID,),
        in_specs=[_row_spec(128), _part_spec(128, 0), _part_spec(128, 1),
                  cnt_spec,
                  _full_spec((128, 128)), _full_spec((1, 128)),
                  _full_spec((128, 128))],
        out_specs=_row_spec(128),
        out_shape=jax.ShapeDtypeStruct((N, 128), jnp.float32),
    )(x, agg1, agg1, inv, w1l, b1r, w1r)

    (agg2,) = _sc_agg128(h1, edges, zf128)

    h2, p3 = pl.pallas_call(
        _tc2_body,
        grid=(_GRID,),
        in_specs=[_row_spec(128), _part_spec(128, 0), _part_spec(128, 1),
                  cnt_spec,
                  _full_spec((128, 64)), _full_spec((1, 64)),
                  _full_spec((128, 64)), _full_spec((64, 128))],
        out_specs=[_row_spec(64), _row_spec(128)],
        out_shape=[jax.ShapeDtypeStruct((N, 64), jnp.float32),
                   jax.ShapeDtypeStruct((N, 128), jnp.float32)],
    )(h1, agg2, agg2, inv, w2l, b2r, w2r, w3l)

    (agg3,) = _sc_agg128(p3, edges, zf128)

    h3 = pl.pallas_call(
        _tc3_body,
        grid=(_GRID,),
        in_specs=[_row_spec(64), _part_spec(128, 0), _part_spec(128, 1),
                  cnt_spec,
                  _full_spec((1, 128)), _full_spec((64, 128))],
        out_specs=_row_spec(128),
        out_shape=jax.ShapeDtypeStruct((N, 128), jnp.float32),
    )(h2, agg3, agg3, inv, b3r, w3r)

    return h3
